# baseline (speedup 1.0000x reference)
import jax
import jax.numpy as jnp
from jax.experimental import pallas as pl
from jax.experimental.pallas import tpu as pltpu

_K = 512
_NUM_ITERS = 4
_TN = 2048


def _kmeans_body(x_ref, xt_ref, cidx_ref, ridx_ref, centers_ref, assign_ref):
    N, D = x_ref.shape[1], x_ref.shape[2]
    nt = N // _TN
    cidx = cidx_ref[0]
    ridx_all = ridx_ref[0]

    def gather(idx_col):
        acc = jnp.zeros((_K, D), jnp.float32)
        for t in range(nt):
            xt = x_ref[0, pl.ds(t * _TN, _TN), :]
            iota_n = jax.lax.broadcasted_iota(jnp.int32, (_K, _TN), 1) + t * _TN
            oh = (idx_col == iota_n).astype(jnp.float32)
            acc = acc + jax.lax.dot_general(
                oh, xt, (((1,), (0,)), ((), ())),
                preferred_element_type=jnp.float32,
                precision=jax.lax.Precision.HIGHEST)
        return acc

    centers = gather(cidx)

    for i in range(_NUM_ITERS):
        c2 = jnp.sum(centers * centers, axis=1, keepdims=True)
        ridx = ridx_all[:, i:i + 1]
        sums = jnp.zeros((_K, D), jnp.float32)
        repl = jnp.zeros((_K, D), jnp.float32)
        counts = jnp.zeros((_K, 1), jnp.float32)
        for t in range(nt):
            xt = x_ref[0, pl.ds(t * _TN, _TN), :]
            xtt = xt_ref[0, :, pl.ds(t * _TN, _TN)]
            xct = jax.lax.dot_general(
                centers.astype(jnp.bfloat16), xtt.astype(jnp.bfloat16),
                (((1,), (0,)), ((), ())),
                preferred_element_type=jnp.float32)
            d2 = c2 - 2.0 * xct
            m = jnp.min(d2, axis=0, keepdims=True)
            iota_k = jax.lax.broadcasted_iota(jnp.int32, (_K, _TN), 0)
            assign = jnp.min(jnp.where(d2 == m, iota_k, _K),
                             axis=0, keepdims=True)
            if i == _NUM_ITERS - 1:
                assign_ref[0, :, pl.ds(t * _TN, _TN)] = assign
            oh = (iota_k == assign).astype(jnp.float32)
            sums = sums + jax.lax.dot_general(
                oh, xt, (((1,), (0,)), ((), ())),
                preferred_element_type=jnp.float32,
                precision=jax.lax.Precision.HIGHEST)
            counts = counts + jnp.sum(oh, axis=1, keepdims=True)
            iota_n = jax.lax.broadcasted_iota(jnp.int32, (_K, _TN), 1) + t * _TN
            ohr = (ridx == iota_n).astype(jnp.float32)
            repl = repl + jax.lax.dot_general(
                ohr, xt, (((1,), (0,)), ((), ())),
                preferred_element_type=jnp.float32,
                precision=jax.lax.Precision.HIGHEST)
        centers = jnp.where(counts == 0.0, repl,
                            sums / jnp.maximum(counts, 1.0))
    centers_ref[0] = centers


def kernel(x):
    B, N, D = x.shape
    key = jax.random.key(42)
    k_init, key = jax.random.split(key)
    random_order = jnp.argsort(jax.random.uniform(k_init, (B, N)), axis=1)
    cidx = random_order[:, :_K].astype(jnp.int32)[:, :, None]
    ridx = jnp.stack(
        [jax.random.randint(jax.random.fold_in(key, i), (B, _K), 0, N)
         for i in range(_NUM_ITERS)], axis=-1).astype(jnp.int32)
    xt = jnp.swapaxes(x, 1, 2)
    centers, assign = pl.pallas_call(
        _kmeans_body,
        grid=(B,),
        in_specs=[
            pl.BlockSpec((1, N, D), lambda b: (b, 0, 0)),
            pl.BlockSpec((1, D, N), lambda b: (b, 0, 0)),
            pl.BlockSpec((1, _K, 1), lambda b: (b, 0, 0)),
            pl.BlockSpec((1, _K, _NUM_ITERS), lambda b: (b, 0, 0)),
        ],
        out_specs=[
            pl.BlockSpec((1, _K, D), lambda b: (b, 0, 0)),
            pl.BlockSpec((1, 1, N), lambda b: (b, 0, 0)),
        ],
        out_shape=[
            jax.ShapeDtypeStruct((B, _K, D), jnp.float32),
            jax.ShapeDtypeStruct((B, 1, N), jnp.int32),
        ],
    )(x, xt, cidx, ridx)
    return centers, assign.reshape(B, N)


# speedup vs baseline: 1.0131x; 1.0131x over previous
import jax
import jax.numpy as jnp
from jax.experimental import pallas as pl
from jax.experimental.pallas import tpu as pltpu

_K = 512
_NUM_ITERS = 4
_TN = 2048


def _kmeans_body(x_ref, xt_ref, cidx_ref, ridx_ref, centers_ref, assign_ref):
    N, D = x_ref.shape[1], x_ref.shape[2] - 1
    nt = N // _TN
    cidx = cidx_ref[0]
    ridx_all = ridx_ref[0]

    def gather(idx_col):
        acc = jnp.zeros((_K, D + 1), jnp.float32)
        for t in range(nt):
            xt = x_ref[0, pl.ds(t * _TN, _TN), :]
            iota_n = jax.lax.broadcasted_iota(jnp.int32, (_K, _TN), 1) + t * _TN
            oh = (idx_col == iota_n).astype(jnp.float32)
            acc = acc + jax.lax.dot_general(
                oh, xt, (((1,), (0,)), ((), ())),
                preferred_element_type=jnp.float32,
                precision=jax.lax.Precision.HIGHEST)
        return acc

    centers = gather(cidx)[:, :D]

    for i in range(_NUM_ITERS):
        c2 = jnp.sum(centers * centers, axis=1, keepdims=True)
        ridx = ridx_all[:, i:i + 1]
        sums = jnp.zeros((_K, D + 1), jnp.float32)
        repl = jnp.zeros((_K, D + 1), jnp.float32)
        for t in range(nt):
            xt = x_ref[0, pl.ds(t * _TN, _TN), :]
            xtt = xt_ref[0, :, pl.ds(t * _TN, _TN)]
            xct = jax.lax.dot_general(
                centers.astype(jnp.bfloat16), xtt.astype(jnp.bfloat16),
                (((1,), (0,)), ((), ())),
                preferred_element_type=jnp.float32)
            d2 = c2 - 2.0 * xct
            m = jnp.min(d2, axis=0, keepdims=True)
            iota_k = jax.lax.broadcasted_iota(jnp.int32, (_K, _TN), 0)
            assign = jnp.min(jnp.where(d2 == m, iota_k, _K),
                             axis=0, keepdims=True)
            if i == _NUM_ITERS - 1:
                assign_ref[0, :, pl.ds(t * _TN, _TN)] = assign
            oh = (iota_k == assign).astype(jnp.float32)
            sums = sums + jax.lax.dot_general(
                oh, xt, (((1,), (0,)), ((), ())),
                preferred_element_type=jnp.float32,
                precision=jax.lax.Precision.HIGHEST)
            iota_n = jax.lax.broadcasted_iota(jnp.int32, (_K, _TN), 1) + t * _TN
            ohr = (ridx == iota_n).astype(jnp.float32)
            repl = repl + jax.lax.dot_general(
                ohr, xt, (((1,), (0,)), ((), ())),
                preferred_element_type=jnp.float32,
                precision=jax.lax.Precision.HIGHEST)
        counts = sums[:, D:D + 1]
        centers = jnp.where(counts == 0.0, repl[:, :D],
                            sums[:, :D] / jnp.maximum(counts, 1.0))
    centers_ref[0] = centers


def kernel(x):
    B, N, D = x.shape
    key = jax.random.key(42)
    k_init, key = jax.random.split(key)
    random_order = jnp.argsort(jax.random.uniform(k_init, (B, N)), axis=1)
    cidx = random_order[:, :_K].astype(jnp.int32)[:, :, None]
    ridx = jnp.stack(
        [jax.random.randint(jax.random.fold_in(key, i), (B, _K), 0, N)
         for i in range(_NUM_ITERS)], axis=-1).astype(jnp.int32)
    xt = jnp.swapaxes(x, 1, 2)
    x_aug = jnp.concatenate(
        [x, jnp.ones((B, N, 1), jnp.float32)], axis=2)
    centers, assign = pl.pallas_call(
        _kmeans_body,
        grid=(B,),
        compiler_params=pltpu.CompilerParams(
            dimension_semantics=("parallel",)),
        in_specs=[
            pl.BlockSpec((1, N, D + 1), lambda b: (b, 0, 0)),
            pl.BlockSpec((1, D, N), lambda b: (b, 0, 0)),
            pl.BlockSpec((1, _K, 1), lambda b: (b, 0, 0)),
            pl.BlockSpec((1, _K, _NUM_ITERS), lambda b: (b, 0, 0)),
        ],
        out_specs=[
            pl.BlockSpec((1, _K, D), lambda b: (b, 0, 0)),
            pl.BlockSpec((1, 1, N), lambda b: (b, 0, 0)),
        ],
        out_shape=[
            jax.ShapeDtypeStruct((B, _K, D), jnp.float32),
            jax.ShapeDtypeStruct((B, 1, N), jnp.int32),
        ],
    )(x_aug, xt, cidx, ridx)
    return centers, assign.reshape(B, N)


# speedup vs baseline: 3.0136x; 2.9747x over previous
import jax
import jax.numpy as jnp
from jax.experimental import pallas as pl
from jax.experimental.pallas import tpu as pltpu

_K = 512
_NUM_ITERS = 4
_TN = 2048


def _kmeans_body(xh_ref, xtb_ref, cidx_ref, ridx_ref, centers_ref, assign_ref):
    N = xh_ref.shape[1]
    da = xh_ref.shape[2] // 3
    D = da - 1
    nt = N // _TN
    cidx = cidx_ref[0]
    ridx_all = ridx_ref[0]

    iota_k = jax.lax.broadcasted_iota(jnp.int32, (_K, _TN), 0)
    iota_n0 = jax.lax.broadcasted_iota(jnp.int32, (_K, _TN), 1)

    def parts_sum(acc99):
        return (acc99[:, :da] + acc99[:, da:2 * da]) + acc99[:, 2 * da:]

    def onehot_dot(oh, t, acc):
        xt3 = xh_ref[0, pl.ds(t * _TN, _TN), :]
        return acc + jax.lax.dot_general(
            oh, xt3, (((1,), (0,)), ((), ())),
            preferred_element_type=jnp.float32)

    def gather(idx_col):
        acc = jnp.zeros((_K, 3 * da), jnp.float32)
        for t in range(nt):
            oh = (idx_col == iota_n0 + t * _TN).astype(jnp.bfloat16)
            acc = onehot_dot(oh, t, acc)
        return parts_sum(acc)

    centers = gather(cidx)[:, :D]

    for i in range(_NUM_ITERS):
        c2 = jnp.sum(centers * centers, axis=1, keepdims=True)
        cb = centers.astype(jnp.bfloat16)
        ridx = ridx_all[:, i:i + 1]
        sums = jnp.zeros((_K, 3 * da), jnp.float32)
        repl = jnp.zeros((_K, 3 * da), jnp.float32)
        for t in range(nt):
            xtt = xtb_ref[0, :, pl.ds(t * _TN, _TN)]
            xct = jax.lax.dot_general(
                cb, xtt, (((1,), (0,)), ((), ())),
                preferred_element_type=jnp.float32)
            d2 = c2 - 2.0 * xct
            m = jnp.min(d2, axis=0, keepdims=True)
            masked = jnp.where(d2 == m, iota_k, _K)
            assign = jnp.min(masked, axis=0, keepdims=True)
            if i == _NUM_ITERS - 1:
                assign_ref[0, :, pl.ds(t * _TN, _TN)] = assign
            oh = (iota_k == assign).astype(jnp.bfloat16)
            sums = onehot_dot(oh, t, sums)
            ohr = (ridx == iota_n0 + t * _TN).astype(jnp.bfloat16)
            repl = onehot_dot(ohr, t, repl)
        sums = parts_sum(sums)
        counts = sums[:, D:da]
        centers = jnp.where(counts == 0.0, parts_sum(repl)[:, :D],
                            sums[:, :D] / jnp.maximum(counts, 1.0))
    centers_ref[0] = centers


def kernel(x):
    B, N, D = x.shape
    key = jax.random.key(42)
    k_init, key = jax.random.split(key)
    random_order = jnp.argsort(jax.random.uniform(k_init, (B, N)), axis=1)
    cidx = random_order[:, :_K].astype(jnp.int32)[:, :, None]
    ridx = jnp.stack(
        [jax.random.randint(jax.random.fold_in(key, i), (B, _K), 0, N)
         for i in range(_NUM_ITERS)], axis=-1).astype(jnp.int32)
    x_aug = jnp.concatenate(
        [x, jnp.ones((B, N, 1), jnp.float32)], axis=2)
    h1 = x_aug.astype(jnp.bfloat16)
    r1 = x_aug - h1.astype(jnp.float32)
    h2 = r1.astype(jnp.bfloat16)
    h3 = (r1 - h2.astype(jnp.float32)).astype(jnp.bfloat16)
    xh = jnp.concatenate([h1, h2, h3], axis=2)
    xtb = jnp.swapaxes(x.astype(jnp.bfloat16), 1, 2)
    centers, assign = pl.pallas_call(
        _kmeans_body,
        grid=(B,),
        compiler_params=pltpu.CompilerParams(
            dimension_semantics=("parallel",)),
        in_specs=[
            pl.BlockSpec((1, N, 3 * (D + 1)), lambda b: (b, 0, 0)),
            pl.BlockSpec((1, D, N), lambda b: (b, 0, 0)),
            pl.BlockSpec((1, _K, 1), lambda b: (b, 0, 0)),
            pl.BlockSpec((1, _K, _NUM_ITERS), lambda b: (b, 0, 0)),
        ],
        out_specs=[
            pl.BlockSpec((1, _K, D), lambda b: (b, 0, 0)),
            pl.BlockSpec((1, 1, N), lambda b: (b, 0, 0)),
        ],
        out_shape=[
            jax.ShapeDtypeStruct((B, _K, D), jnp.float32),
            jax.ShapeDtypeStruct((B, 1, N), jnp.int32),
        ],
    )(xh, xtb, cidx, ridx)
    return centers, assign.reshape(B, N)
